# Initial kernel scaffold; baseline (speedup 1.0000x reference)
import jax
import jax.numpy as jnp
from jax.experimental import pallas as pl


def kernel(x, edge_index, edge_weight, W1, b1, gamma1, beta1, W2, b2):
    raise NotImplementedError("write your pallas kernel here")


# speedup vs baseline: 18.7220x; 18.7220x over previous
import functools

import jax
import jax.numpy as jnp
from jax import lax
from jax.experimental import pallas as pl
from jax.experimental.pallas import tpu as pltpu
from jax.experimental.pallas import tpu_sc as plsc

NC = 2
NS = 16
NW = NC * NS

K = 80
DEG_W = 8


def _sc_mesh():
    return plsc.VectorSubcoreMesh(
        core_axis_name="c", subcore_axis_name="s", num_cores=NC,
        num_subcores=NS)


def _make_deg_kernel(n, e):
    nch = e // (NW * K)
    rows_per_tile = n // NS

    @functools.partial(
        pl.kernel,
        out_type=jax.ShapeDtypeStruct((NC, n, DEG_W), jnp.float32),
        mesh=_sc_mesh(),
        compiler_params=pltpu.CompilerParams(use_tc_tiling_on_sc=False),
        scratch_types=[
            pltpu.VMEM((nch, K), jnp.int32),
            pltpu.VMEM((K, DEG_W), jnp.float32),
            pltpu.VMEM_SHARED((n, DEG_W), jnp.float32),
        ],
    )
    def deg_kernel(col_hbm, ones_hbm, zeros_hbm, out_hbm, idx_c, ones_v,
                   acc):
        cid = lax.axis_index("c")
        sid = lax.axis_index("s")
        wid = sid * NC + cid
        pltpu.sync_copy(col_hbm.at[wid], idx_c)
        pltpu.sync_copy(ones_hbm, ones_v)
        sl = pl.ds(sid * rows_per_tile, rows_per_tile)
        pltpu.sync_copy(zeros_hbm, acc.at[sl])
        plsc.subcore_barrier()

        def step(j, carry):
            pltpu.sync_copy(ones_v, acc.at[idx_c.at[j]], add=True)
            return carry

        lax.fori_loop(0, nch, step, 0)
        plsc.subcore_barrier()
        pltpu.sync_copy(acc.at[sl], out_hbm.at[cid, sl])

    return deg_kernel


def _make_agg_kernel(n, e, d):
    nch = e // (NW * K)
    rows_per_tile = n // NS

    @functools.partial(
        pl.kernel,
        out_type=jax.ShapeDtypeStruct((NC, n, d), jnp.float32),
        mesh=_sc_mesh(),
        compiler_params=pltpu.CompilerParams(use_tc_tiling_on_sc=False),
        scratch_types=[
            pltpu.VMEM((nch, K), jnp.int32),
            pltpu.VMEM((nch, K), jnp.int32),
            pltpu.VMEM((K, d), jnp.float32),
            pltpu.VMEM((K, d), jnp.float32),
            pltpu.VMEM_SHARED((n, d), jnp.float32),
            pltpu.SemaphoreType.DMA,
            pltpu.SemaphoreType.DMA,
        ],
    )
    def agg_kernel(h_hbm, row_hbm, col_hbm, zeros_hbm, out_hbm, idx_r,
                   idx_c, buf0, buf1, acc, sem_g, sem_s):
        cid = lax.axis_index("c")
        sid = lax.axis_index("s")
        wid = sid * NC + cid
        pltpu.sync_copy(row_hbm.at[wid], idx_r)
        pltpu.sync_copy(col_hbm.at[wid], idx_c)
        sl = pl.ds(sid * rows_per_tile, rows_per_tile)
        pltpu.sync_copy(zeros_hbm, acc.at[sl])
        plsc.subcore_barrier()

        def step(i, carry):
            j0 = 2 * i
            j1 = j0 + 1
            g0 = pltpu.async_copy(h_hbm.at[idx_r.at[j0]], buf0, sem_g)
            g0.wait()
            s0 = pltpu.async_copy(buf0, acc.at[idx_c.at[j0]], sem_s,
                                  add=True)
            g1 = pltpu.async_copy(h_hbm.at[idx_r.at[j1]], buf1, sem_g)
            g1.wait()
            s0.wait()
            s1 = pltpu.async_copy(buf1, acc.at[idx_c.at[j1]], sem_s,
                                  add=True)
            s1.wait()
            return carry

        lax.fori_loop(0, nch // 2, step, 0)
        if nch % 2:
            j = nch - 1
            pltpu.async_copy(h_hbm.at[idx_r.at[j]], buf0, sem_g).wait()
            pltpu.async_copy(buf0, acc.at[idx_c.at[j]], sem_s,
                             add=True).wait()
        plsc.subcore_barrier()
        pltpu.sync_copy(acc.at[sl], out_hbm.at[cid, sl])

    return agg_kernel


def _mm1_body(x_ref, w_ref, degp_ref, h1sa_ref, h1sb_ref, dinv_ref):
    deg = degp_ref[0, :, 0:1] + degp_ref[1, :, 0:1] + 1.0
    dinv = lax.rsqrt(deg)
    h = jnp.dot(x_ref[...], w_ref[...], preferred_element_type=jnp.float32)
    hs = dinv * h
    half = hs.shape[1] // 2
    h1sa_ref[...] = hs[:, :half]
    h1sb_ref[...] = hs[:, half:]
    dinv_ref[...] = jnp.broadcast_to(dinv, dinv_ref.shape)


def _mid_body(pa_ref, pb_ref, h1sa_ref, h1sb_ref, dinv_ref, b1_ref, g_ref,
              be_ref, w2_ref, h2s_ref):
    dinv = dinv_ref[:, 0:1]
    s = jnp.concatenate([pa_ref[0] + pa_ref[1] + h1sa_ref[...],
                         pb_ref[0] + pb_ref[1] + h1sb_ref[...]], axis=-1)
    t = dinv * s + b1_ref[...]
    m = jnp.mean(t, axis=-1, keepdims=True)
    v = jnp.mean((t - m) ** 2, axis=-1, keepdims=True)
    t = (t - m) * lax.rsqrt(v + 1e-6) * g_ref[...] + be_ref[...]
    a = jnp.maximum(t, 0.0)
    h2 = jnp.dot(a, w2_ref[...], preferred_element_type=jnp.float32)
    h2s_ref[...] = dinv * h2


def _final_body(p_ref, h2s_ref, dinv_ref, b2_ref, out_ref):
    dinv = dinv_ref[:, 0:1]
    out_ref[...] = dinv * (p_ref[0] + p_ref[1] + h2s_ref[...]) + b2_ref[...]


def kernel(x, edge_index, edge_weight, W1, b1, gamma1, beta1, W2, b2):
    n0, d = x.shape
    hdim = W1.shape[1]
    ncls = W2.shape[1]
    e = edge_index.shape[1]
    nch = e // (NW * K)
    n = ((n0 + NS * 8 - 1) // (NS * 8)) * (NS * 8)
    n = max(n, 10240)
    rows_per_tile = n // NS
    d2 = 64

    x = jnp.pad(x, ((0, n - n0), (0, 0)))
    row = edge_index[0].reshape(NW, nch, K)
    col = edge_index[1].reshape(NW, nch, K)
    ones_deg = jnp.ones((K, DEG_W), jnp.float32)
    zeros_deg = jnp.zeros((rows_per_tile, DEG_W), jnp.float32)
    zeros_2 = jnp.zeros((rows_per_tile, d2), jnp.float32)
    W2p = jnp.pad(W2, ((0, 0), (0, d2 - ncls)))
    b2p = jnp.pad(b2, (0, d2 - ncls)).reshape(1, d2)

    degp = _make_deg_kernel(n, e)(col, ones_deg, zeros_deg)

    R = 640
    grid = (n // R,)
    half = hdim // 2
    h1sa, h1sb, dinv = pl.pallas_call(
        _mm1_body,
        grid=grid,
        in_specs=[
            pl.BlockSpec((R, d), lambda i: (i, 0)),
            pl.BlockSpec((d, hdim), lambda i: (0, 0)),
            pl.BlockSpec((NC, R, DEG_W), lambda i: (0, i, 0)),
        ],
        out_specs=[
            pl.BlockSpec((R, half), lambda i: (i, 0)),
            pl.BlockSpec((R, half), lambda i: (i, 0)),
            pl.BlockSpec((R, 8), lambda i: (i, 0)),
        ],
        out_shape=[
            jax.ShapeDtypeStruct((n, half), jnp.float32),
            jax.ShapeDtypeStruct((n, half), jnp.float32),
            jax.ShapeDtypeStruct((n, 8), jnp.float32),
        ],
    )(x, W1, degp)

    agg64 = _make_agg_kernel(n, e, half)
    p1a = agg64(h1sa, row, col, zeros_2)
    p1b = agg64(h1sb, row, col, zeros_2)

    h2s = pl.pallas_call(
        _mid_body,
        grid=grid,
        in_specs=[
            pl.BlockSpec((NC, R, half), lambda i: (0, i, 0)),
            pl.BlockSpec((NC, R, half), lambda i: (0, i, 0)),
            pl.BlockSpec((R, half), lambda i: (i, 0)),
            pl.BlockSpec((R, half), lambda i: (i, 0)),
            pl.BlockSpec((R, 8), lambda i: (i, 0)),
            pl.BlockSpec((1, hdim), lambda i: (0, 0)),
            pl.BlockSpec((1, hdim), lambda i: (0, 0)),
            pl.BlockSpec((1, hdim), lambda i: (0, 0)),
            pl.BlockSpec((hdim, d2), lambda i: (0, 0)),
        ],
        out_specs=pl.BlockSpec((R, d2), lambda i: (i, 0)),
        out_shape=jax.ShapeDtypeStruct((n, d2), jnp.float32),
    )(p1a, p1b, h1sa, h1sb, dinv, b1.reshape(1, hdim),
      gamma1.reshape(1, hdim), beta1.reshape(1, hdim), W2p)

    p2 = agg64(h2s, row, col, zeros_2)

    out = pl.pallas_call(
        _final_body,
        grid=grid,
        in_specs=[
            pl.BlockSpec((NC, R, d2), lambda i: (0, i, 0)),
            pl.BlockSpec((R, d2), lambda i: (i, 0)),
            pl.BlockSpec((R, 8), lambda i: (i, 0)),
            pl.BlockSpec((1, d2), lambda i: (0, 0)),
        ],
        out_specs=pl.BlockSpec((R, d2), lambda i: (i, 0)),
        out_shape=jax.ShapeDtypeStruct((n, d2), jnp.float32),
    )(p2, h2s, dinv, b2p)

    return out[:n0, :ncls]


# speedup vs baseline: 20.4369x; 1.0916x over previous
import functools

import jax
import jax.numpy as jnp
from jax import lax
from jax.experimental import pallas as pl
from jax.experimental.pallas import tpu as pltpu
from jax.experimental.pallas import tpu_sc as plsc

NC = 2
NS = 16
NW = NC * NS

K = 100
DEG_W = 8


def _sc_mesh():
    return plsc.VectorSubcoreMesh(
        core_axis_name="c", subcore_axis_name="s", num_cores=NC,
        num_subcores=NS)


def _make_deg_kernel(n, e):
    nch = e // (NW * K)
    rows_per_tile = n // NS

    @functools.partial(
        pl.kernel,
        out_type=jax.ShapeDtypeStruct((NC, n, DEG_W), jnp.float32),
        mesh=_sc_mesh(),
        compiler_params=pltpu.CompilerParams(use_tc_tiling_on_sc=False),
        scratch_types=[
            pltpu.VMEM((nch, K), jnp.int32),
            pltpu.VMEM((K, DEG_W), jnp.float32),
            pltpu.VMEM_SHARED((n, DEG_W), jnp.float32),
        ],
    )
    def deg_kernel(col_hbm, ones_hbm, zeros_hbm, out_hbm, idx_c, ones_v,
                   acc):
        cid = lax.axis_index("c")
        sid = lax.axis_index("s")
        wid = sid * NC + cid
        pltpu.sync_copy(col_hbm.at[wid], idx_c)
        pltpu.sync_copy(ones_hbm, ones_v)
        sl = pl.ds(sid * rows_per_tile, rows_per_tile)
        pltpu.sync_copy(zeros_hbm, acc.at[sl])
        plsc.subcore_barrier()

        def step(j, carry):
            pltpu.sync_copy(ones_v, acc.at[idx_c.at[j]], add=True)
            return carry

        lax.fori_loop(0, nch, step, 0)
        plsc.subcore_barrier()
        pltpu.sync_copy(acc.at[sl], out_hbm.at[cid, sl])

    return deg_kernel


def _make_agg_kernel(n, e, d, feature_split):
    tiles = NS if feature_split else NW
    nch = e // (tiles * K)
    rows_per_tile = n // NS

    @functools.partial(
        pl.kernel,
        out_type=jax.ShapeDtypeStruct((NC, n, d), jnp.float32),
        mesh=_sc_mesh(),
        compiler_params=pltpu.CompilerParams(use_tc_tiling_on_sc=False),
        scratch_types=[
            pltpu.VMEM((nch, K), jnp.int32),
            pltpu.VMEM((nch, K), jnp.int32),
            pltpu.VMEM((K, d), jnp.float32),
            pltpu.VMEM((K, d), jnp.float32),
            pltpu.VMEM_SHARED((n, d), jnp.float32),
            pltpu.SemaphoreType.DMA,
            pltpu.SemaphoreType.DMA,
        ],
    )
    def agg_kernel(h_hbm, row_hbm, col_hbm, zeros_hbm, out_hbm, idx_r,
                   idx_c, buf0, buf1, acc, sem_g, sem_s):
        cid = lax.axis_index("c")
        sid = lax.axis_index("s")
        if feature_split:
            pltpu.sync_copy(row_hbm.at[cid, sid], idx_r)
            pltpu.sync_copy(col_hbm.at[sid], idx_c)
        else:
            wid = sid * NC + cid
            pltpu.sync_copy(row_hbm.at[wid], idx_r)
            pltpu.sync_copy(col_hbm.at[wid], idx_c)
        sl = pl.ds(sid * rows_per_tile, rows_per_tile)
        pltpu.sync_copy(zeros_hbm, acc.at[sl])
        plsc.subcore_barrier()

        def step(i, carry):
            j0 = 2 * i
            j1 = j0 + 1
            g0 = pltpu.async_copy(h_hbm.at[idx_r.at[j0]], buf0, sem_g)
            g0.wait()
            s0 = pltpu.async_copy(buf0, acc.at[idx_c.at[j0]], sem_s,
                                  add=True)
            g1 = pltpu.async_copy(h_hbm.at[idx_r.at[j1]], buf1, sem_g)
            g1.wait()
            s0.wait()
            s1 = pltpu.async_copy(buf1, acc.at[idx_c.at[j1]], sem_s,
                                  add=True)
            s1.wait()
            return carry

        lax.fori_loop(0, nch // 2, step, 0)
        plsc.subcore_barrier()
        pltpu.sync_copy(acc.at[sl], out_hbm.at[cid, sl])

    return agg_kernel


def _mm1_body(x_ref, w_ref, degp_ref, hs_ref, dinv_ref):
    deg = degp_ref[0, :, 0:1] + degp_ref[1, :, 0:1] + 1.0
    dinv = lax.rsqrt(deg)
    h = jnp.dot(x_ref[...], w_ref[...], preferred_element_type=jnp.float32)
    hs = dinv * h
    half = hs.shape[1] // 2
    hs_ref[0] = hs[:, :half]
    hs_ref[1] = hs[:, half:]
    dinv_ref[...] = jnp.broadcast_to(dinv, dinv_ref.shape)


def _mid_body(p_ref, hs_ref, dinv_ref, b1_ref, g_ref, be_ref, w2_ref,
              h2s_ref):
    dinv = dinv_ref[:, 0:1]
    s = jnp.concatenate([p_ref[0] + hs_ref[0], p_ref[1] + hs_ref[1]],
                        axis=-1)
    t = dinv * s + b1_ref[...]
    m = jnp.mean(t, axis=-1, keepdims=True)
    v = jnp.mean((t - m) ** 2, axis=-1, keepdims=True)
    t = (t - m) * lax.rsqrt(v + 1e-6) * g_ref[...] + be_ref[...]
    a = jnp.maximum(t, 0.0)
    h2 = jnp.dot(a, w2_ref[...], preferred_element_type=jnp.float32)
    h2s_ref[...] = dinv * h2


def _final_body(p_ref, h2s_ref, dinv_ref, b2_ref, out_ref):
    dinv = dinv_ref[:, 0:1]
    out_ref[...] = dinv * (p_ref[0] + p_ref[1] + h2s_ref[...]) + b2_ref[...]


def kernel(x, edge_index, edge_weight, W1, b1, gamma1, beta1, W2, b2):
    n0, d = x.shape
    hdim = W1.shape[1]
    ncls = W2.shape[1]
    e = edge_index.shape[1]
    n = ((n0 + NS * 8 - 1) // (NS * 8)) * (NS * 8)
    n = max(n, 10240)
    rows_per_tile = n // NS
    d2 = 64
    half = hdim // 2

    x = jnp.pad(x, ((0, n - n0), (0, 0)))
    row = edge_index[0]
    col = edge_index[1]
    nch_e = e // (NW * K)
    row_e = row.reshape(NW, nch_e, K)
    col_e = col.reshape(NW, nch_e, K)
    nch_f = e // (NS * K)
    row_f = jnp.stack([row, row + n]).reshape(NC, NS, nch_f, K)
    col_f = col.reshape(NS, nch_f, K)

    ones_deg = jnp.ones((K, DEG_W), jnp.float32)
    zeros_deg = jnp.zeros((rows_per_tile, DEG_W), jnp.float32)
    zeros_2 = jnp.zeros((rows_per_tile, d2), jnp.float32)
    W2p = jnp.pad(W2, ((0, 0), (0, d2 - ncls)))
    b2p = jnp.pad(b2, (0, d2 - ncls)).reshape(1, d2)

    degp = _make_deg_kernel(n, e)(col_e, ones_deg, zeros_deg)

    R = 640
    grid = (n // R,)
    hs12, dinv = pl.pallas_call(
        _mm1_body,
        grid=grid,
        in_specs=[
            pl.BlockSpec((R, d), lambda i: (i, 0)),
            pl.BlockSpec((d, hdim), lambda i: (0, 0)),
            pl.BlockSpec((NC, R, DEG_W), lambda i: (0, i, 0)),
        ],
        out_specs=[
            pl.BlockSpec((NC, R, half), lambda i: (0, i, 0)),
            pl.BlockSpec((R, 8), lambda i: (i, 0)),
        ],
        out_shape=[
            jax.ShapeDtypeStruct((NC, n, half), jnp.float32),
            jax.ShapeDtypeStruct((n, 8), jnp.float32),
        ],
    )(x, W1, degp)

    p1 = _make_agg_kernel(n, e, half, True)(
        hs12.reshape(NC * n, half), row_f, col_f, zeros_2)

    h2s = pl.pallas_call(
        _mid_body,
        grid=grid,
        in_specs=[
            pl.BlockSpec((NC, R, half), lambda i: (0, i, 0)),
            pl.BlockSpec((NC, R, half), lambda i: (0, i, 0)),
            pl.BlockSpec((R, 8), lambda i: (i, 0)),
            pl.BlockSpec((1, hdim), lambda i: (0, 0)),
            pl.BlockSpec((1, hdim), lambda i: (0, 0)),
            pl.BlockSpec((1, hdim), lambda i: (0, 0)),
            pl.BlockSpec((hdim, d2), lambda i: (0, 0)),
        ],
        out_specs=pl.BlockSpec((R, d2), lambda i: (i, 0)),
        out_shape=jax.ShapeDtypeStruct((n, d2), jnp.float32),
    )(p1, hs12, dinv, b1.reshape(1, hdim), gamma1.reshape(1, hdim),
      beta1.reshape(1, hdim), W2p)

    p2 = _make_agg_kernel(n, e, d2, False)(h2s, row_e, col_e, zeros_2)

    out = pl.pallas_call(
        _final_body,
        grid=grid,
        in_specs=[
            pl.BlockSpec((NC, R, d2), lambda i: (0, i, 0)),
            pl.BlockSpec((R, d2), lambda i: (i, 0)),
            pl.BlockSpec((R, 8), lambda i: (i, 0)),
            pl.BlockSpec((1, d2), lambda i: (0, 0)),
        ],
        out_specs=pl.BlockSpec((R, d2), lambda i: (i, 0)),
        out_shape=jax.ShapeDtypeStruct((n, d2), jnp.float32),
    )(p2, h2s, dinv, b2p)

    return out[:n0, :ncls]


# speedup vs baseline: 29.0351x; 1.4207x over previous
import functools

import jax
import jax.numpy as jnp
from jax import lax
from jax.experimental import pallas as pl
from jax.experimental.pallas import tpu as pltpu
from jax.experimental.pallas import tpu_sc as plsc

NC = 2
NS = 16
NW = NC * NS

K = 100
DEG_W = 8


def _sc_mesh():
    return plsc.VectorSubcoreMesh(
        core_axis_name="c", subcore_axis_name="s", num_cores=NC,
        num_subcores=NS)


def _make_deg_kernel(n, e):
    nch = e // (NW * K)
    rows_per_tile = n // NS

    @functools.partial(
        pl.kernel,
        out_type=jax.ShapeDtypeStruct((NC, n, DEG_W), jnp.float32),
        mesh=_sc_mesh(),
        compiler_params=pltpu.CompilerParams(use_tc_tiling_on_sc=False),
        scratch_types=[
            pltpu.VMEM((nch, K), jnp.int32),
            pltpu.VMEM((K, DEG_W), jnp.float32),
            pltpu.VMEM_SHARED((n, DEG_W), jnp.float32),
        ],
    )
    def deg_kernel(col_hbm, ones_hbm, zeros_hbm, out_hbm, idx_c, ones_v,
                   acc):
        cid = lax.axis_index("c")
        sid = lax.axis_index("s")
        wid = sid * NC + cid
        pltpu.sync_copy(col_hbm.at[wid], idx_c)
        pltpu.sync_copy(ones_hbm, ones_v)
        sl = pl.ds(sid * rows_per_tile, rows_per_tile)
        pltpu.sync_copy(zeros_hbm, acc.at[sl])
        plsc.subcore_barrier()

        def step(j, carry):
            pltpu.sync_copy(ones_v, acc.at[idx_c.at[j]], add=True)
            return carry

        lax.fori_loop(0, nch, step, 0)
        plsc.subcore_barrier()
        pltpu.sync_copy(acc.at[sl], out_hbm.at[cid, sl])

    return deg_kernel


def _make_agg_kernel(n, e, d, feature_split):
    tiles = NS if feature_split else NW
    nch = e // (tiles * K)
    rows_per_tile = n // NS

    @functools.partial(
        pl.kernel,
        out_type=jax.ShapeDtypeStruct((NC, n, d), jnp.float32),
        mesh=_sc_mesh(),
        compiler_params=pltpu.CompilerParams(use_tc_tiling_on_sc=False),
        scratch_types=(
            [pltpu.VMEM((nch, K), jnp.int32)] * 2
            + [pltpu.VMEM((K, d), jnp.float32)] * 6
            + [pltpu.VMEM_SHARED((n, d), jnp.float32)]
            + [pltpu.SemaphoreType.DMA] * 4
        ),
    )
    def agg_kernel(h_hbm, row_hbm, col_hbm, zeros_hbm, out_hbm, idx_r,
                   idx_c, a0, a1, a2, b0, b1, b2, acc, sga, sgb,
                   ssa, ssb):
        cid = lax.axis_index("c")
        sid = lax.axis_index("s")
        if feature_split:
            pltpu.sync_copy(row_hbm.at[cid, sid], idx_r)
            pltpu.sync_copy(col_hbm.at[sid], idx_c)
        else:
            wid = sid * NC + cid
            pltpu.sync_copy(row_hbm.at[wid], idx_r)
            pltpu.sync_copy(col_hbm.at[wid], idx_c)
        sl = pl.ds(sid * rows_per_tile, rows_per_tile)
        pltpu.sync_copy(zeros_hbm, acc.at[sl])
        plsc.subcore_barrier()

        bufa = (a0, a1, a2)
        bufb = (b0, b1, b2)
        G = 3

        def fire_g(base, bufs, sem):
            for t in range(G):
                pltpu.async_copy(h_hbm.at[idx_r.at[base + t]], bufs[t],
                                 sem)

        def drain_g(base, bufs, sem):
            for t in range(G):
                pltpu.make_async_copy(h_hbm.at[idx_r.at[base + t]],
                                      bufs[t], sem).wait()

        def fire_s(base, bufs, sem):
            for t in range(G):
                pltpu.async_copy(bufs[t], acc.at[idx_c.at[base + t]], sem,
                                 add=True)

        def drain_s(base, bufs, sem):
            for t in range(G):
                pltpu.make_async_copy(bufs[t], acc.at[idx_c.at[base + t]],
                                      sem).wait()

        niter = nch // (2 * G)
        tail = nch - niter * 2 * G

        if niter > 0:
            fire_g(0, bufa, sga)

            def step(i, carry):
                base = i * 2 * G
                drain_g(base, bufa, sga)

                @pl.when(i > 0)
                def _():
                    drain_s(base - G, bufb, ssb)

                fire_g(base + G, bufb, sgb)
                fire_s(base, bufa, ssa)
                drain_g(base + G, bufb, sgb)
                drain_s(base, bufa, ssa)

                @pl.when(i < niter - 1)
                def _():
                    fire_g(base + 2 * G, bufa, sga)

                fire_s(base + G, bufb, ssb)
                return carry

            lax.fori_loop(0, niter, step, 0)
            drain_s(niter * 2 * G - G, bufb, ssb)

        for j in range(nch - tail, nch):
            pltpu.async_copy(h_hbm.at[idx_r.at[j]], a0, sga).wait()
            pltpu.async_copy(a0, acc.at[idx_c.at[j]], ssa, add=True).wait()

        plsc.subcore_barrier()
        pltpu.sync_copy(acc.at[sl], out_hbm.at[cid, sl])

    return agg_kernel


def _mm1_body(x_ref, w_ref, degp_ref, hs_ref, dinv_ref):
    deg = degp_ref[0, :, 0:1] + degp_ref[1, :, 0:1] + 1.0
    dinv = lax.rsqrt(deg)
    h = jnp.dot(x_ref[...], w_ref[...], preferred_element_type=jnp.float32)
    hs = dinv * h
    half = hs.shape[1] // 2
    hs_ref[0] = hs[:, :half]
    hs_ref[1] = hs[:, half:]
    dinv_ref[...] = jnp.broadcast_to(dinv, dinv_ref.shape)


def _mid_body(p_ref, hs_ref, dinv_ref, b1_ref, g_ref, be_ref, w2_ref,
              h2s_ref):
    dinv = dinv_ref[:, 0:1]
    s = jnp.concatenate([p_ref[0] + hs_ref[0], p_ref[1] + hs_ref[1]],
                        axis=-1)
    t = dinv * s + b1_ref[...]
    m = jnp.mean(t, axis=-1, keepdims=True)
    v = jnp.mean((t - m) ** 2, axis=-1, keepdims=True)
    t = (t - m) * lax.rsqrt(v + 1e-6) * g_ref[...] + be_ref[...]
    a = jnp.maximum(t, 0.0)
    h2 = jnp.dot(a, w2_ref[...], preferred_element_type=jnp.float32)
    h2s_ref[...] = dinv * h2


def _final_body(p_ref, h2s_ref, dinv_ref, b2_ref, out_ref):
    dinv = dinv_ref[:, 0:1]
    out_ref[...] = dinv * (p_ref[0] + p_ref[1] + h2s_ref[...]) + b2_ref[...]


def kernel(x, edge_index, edge_weight, W1, b1, gamma1, beta1, W2, b2):
    n0, d = x.shape
    hdim = W1.shape[1]
    ncls = W2.shape[1]
    e = edge_index.shape[1]
    n = ((n0 + NS * 8 - 1) // (NS * 8)) * (NS * 8)
    n = max(n, 10240)
    rows_per_tile = n // NS
    d2 = 64
    half = hdim // 2

    x = jnp.pad(x, ((0, n - n0), (0, 0)))
    row = edge_index[0]
    col = edge_index[1]
    nch_e = e // (NW * K)
    row_e = row.reshape(NW, nch_e, K)
    col_e = col.reshape(NW, nch_e, K)
    nch_f = e // (NS * K)
    row_f = jnp.stack([row, row + n]).reshape(NC, NS, nch_f, K)
    col_f = col.reshape(NS, nch_f, K)

    ones_deg = jnp.ones((K, DEG_W), jnp.float32)
    zeros_deg = jnp.zeros((rows_per_tile, DEG_W), jnp.float32)
    zeros_2 = jnp.zeros((rows_per_tile, d2), jnp.float32)
    W2p = jnp.pad(W2, ((0, 0), (0, d2 - ncls)))
    b2p = jnp.pad(b2, (0, d2 - ncls)).reshape(1, d2)

    degp = _make_deg_kernel(n, e)(col_e, ones_deg, zeros_deg)

    R = 640
    grid = (n // R,)
    hs12, dinv = pl.pallas_call(
        _mm1_body,
        grid=grid,
        in_specs=[
            pl.BlockSpec((R, d), lambda i: (i, 0)),
            pl.BlockSpec((d, hdim), lambda i: (0, 0)),
            pl.BlockSpec((NC, R, DEG_W), lambda i: (0, i, 0)),
        ],
        out_specs=[
            pl.BlockSpec((NC, R, half), lambda i: (0, i, 0)),
            pl.BlockSpec((R, 8), lambda i: (i, 0)),
        ],
        out_shape=[
            jax.ShapeDtypeStruct((NC, n, half), jnp.float32),
            jax.ShapeDtypeStruct((n, 8), jnp.float32),
        ],
    )(x, W1, degp)

    p1 = _make_agg_kernel(n, e, half, True)(
        hs12.reshape(NC * n, half), row_f, col_f, zeros_2)

    h2s = pl.pallas_call(
        _mid_body,
        grid=grid,
        in_specs=[
            pl.BlockSpec((NC, R, half), lambda i: (0, i, 0)),
            pl.BlockSpec((NC, R, half), lambda i: (0, i, 0)),
            pl.BlockSpec((R, 8), lambda i: (i, 0)),
            pl.BlockSpec((1, hdim), lambda i: (0, 0)),
            pl.BlockSpec((1, hdim), lambda i: (0, 0)),
            pl.BlockSpec((1, hdim), lambda i: (0, 0)),
            pl.BlockSpec((hdim, d2), lambda i: (0, 0)),
        ],
        out_specs=pl.BlockSpec((R, d2), lambda i: (i, 0)),
        out_shape=jax.ShapeDtypeStruct((n, d2), jnp.float32),
    )(p1, hs12, dinv, b1.reshape(1, hdim), gamma1.reshape(1, hdim),
      beta1.reshape(1, hdim), W2p)

    p2 = _make_agg_kernel(n, e, d2, False)(h2s, row_e, col_e, zeros_2)

    out = pl.pallas_call(
        _final_body,
        grid=grid,
        in_specs=[
            pl.BlockSpec((NC, R, d2), lambda i: (0, i, 0)),
            pl.BlockSpec((R, d2), lambda i: (i, 0)),
            pl.BlockSpec((R, 8), lambda i: (i, 0)),
            pl.BlockSpec((1, d2), lambda i: (0, 0)),
        ],
        out_specs=pl.BlockSpec((R, d2), lambda i: (i, 0)),
        out_shape=jax.ShapeDtypeStruct((n, d2), jnp.float32),
    )(p2, h2s, dinv, b2p)

    return out[:n0, :ncls]


# speedup vs baseline: 29.7941x; 1.0261x over previous
import functools

import jax
import jax.numpy as jnp
from jax import lax
from jax.experimental import pallas as pl
from jax.experimental.pallas import tpu as pltpu
from jax.experimental.pallas import tpu_sc as plsc

NC = 2
NS = 16
NW = NC * NS

K = 100
DEG_W = 8


def _sc_mesh():
    return plsc.VectorSubcoreMesh(
        core_axis_name="c", subcore_axis_name="s", num_cores=NC,
        num_subcores=NS)


def _make_deg_kernel(n, e):
    nch = e // (NW * K)
    rows_per_tile = n // NS

    @functools.partial(
        pl.kernel,
        out_type=jax.ShapeDtypeStruct((NC, n, DEG_W), jnp.float32),
        mesh=_sc_mesh(),
        compiler_params=pltpu.CompilerParams(use_tc_tiling_on_sc=False),
        scratch_types=[
            pltpu.VMEM((nch, K), jnp.int32),
            pltpu.VMEM((K, DEG_W), jnp.float32),
            pltpu.VMEM_SHARED((n, DEG_W), jnp.float32),
        ],
    )
    def deg_kernel(col_hbm, ones_hbm, zeros_hbm, out_hbm, idx_c, ones_v,
                   acc):
        cid = lax.axis_index("c")
        sid = lax.axis_index("s")
        wid = sid * NC + cid
        pltpu.sync_copy(col_hbm.at[wid], idx_c)
        pltpu.sync_copy(ones_hbm, ones_v)
        sl = pl.ds(sid * rows_per_tile, rows_per_tile)
        pltpu.sync_copy(zeros_hbm, acc.at[sl])
        plsc.subcore_barrier()

        def step(j, carry):
            pltpu.sync_copy(ones_v, acc.at[idx_c.at[j]], add=True)
            return carry

        lax.fori_loop(0, nch, step, 0)
        plsc.subcore_barrier()
        pltpu.sync_copy(acc.at[sl], out_hbm.at[cid, sl])

    return deg_kernel


def _make_agg_kernel(n, e, d, feature_split):
    tiles = NS if feature_split else NW
    nch = e // (tiles * K)
    rows_per_tile = n // NS

    @functools.partial(
        pl.kernel,
        out_type=jax.ShapeDtypeStruct((NC, n, d), jnp.float32),
        mesh=_sc_mesh(),
        compiler_params=pltpu.CompilerParams(use_tc_tiling_on_sc=False),
        scratch_types=(
            [pltpu.VMEM((nch, K), jnp.int32)] * 2
            + [pltpu.VMEM((K, d), jnp.float32)] * 6
            + [pltpu.VMEM_SHARED((n, d), jnp.float32)]
            + [pltpu.SemaphoreType.DMA] * 4
        ),
    )
    def agg_kernel(h_hbm, row_hbm, col_hbm, zeros_hbm, out_hbm, idx_r,
                   idx_c, a0, a1, a2, b0, b1, b2, acc, sga, sgb,
                   ssa, ssb):
        cid = lax.axis_index("c")
        sid = lax.axis_index("s")
        if feature_split:
            src_hbm = h_hbm.at[cid]
            pltpu.sync_copy(row_hbm.at[sid], idx_r)
            pltpu.sync_copy(col_hbm.at[sid], idx_c)
        else:
            src_hbm = h_hbm
            wid = sid * NC + cid
            pltpu.sync_copy(row_hbm.at[wid], idx_r)
            pltpu.sync_copy(col_hbm.at[wid], idx_c)
        sl = pl.ds(sid * rows_per_tile, rows_per_tile)
        pltpu.sync_copy(zeros_hbm, acc.at[sl])
        plsc.subcore_barrier()

        bufa = (a0, a1, a2)
        bufb = (b0, b1, b2)
        G = 3

        def fire_g(base, bufs, sem):
            for t in range(G):
                pltpu.async_copy(src_hbm.at[idx_r.at[base + t]], bufs[t],
                                 sem)

        def drain_g(base, bufs, sem):
            for t in range(G):
                pltpu.make_async_copy(src_hbm.at[idx_r.at[base + t]],
                                      bufs[t], sem).wait()

        def fire_s(base, bufs, sem):
            for t in range(G):
                pltpu.async_copy(bufs[t], acc.at[idx_c.at[base + t]], sem,
                                 add=True)

        def drain_s(base, bufs, sem):
            for t in range(G):
                pltpu.make_async_copy(bufs[t], acc.at[idx_c.at[base + t]],
                                      sem).wait()

        niter = nch // (2 * G)
        tail = nch - niter * 2 * G

        if niter > 0:
            fire_g(0, bufa, sga)

            def step(i, carry):
                base = i * 2 * G
                drain_g(base, bufa, sga)

                @pl.when(i > 0)
                def _():
                    drain_s(base - G, bufb, ssb)

                fire_g(base + G, bufb, sgb)
                fire_s(base, bufa, ssa)
                drain_g(base + G, bufb, sgb)
                drain_s(base, bufa, ssa)

                @pl.when(i < niter - 1)
                def _():
                    fire_g(base + 2 * G, bufa, sga)

                fire_s(base + G, bufb, ssb)
                return carry

            lax.fori_loop(0, niter, step, 0)
            drain_s(niter * 2 * G - G, bufb, ssb)

        for j in range(nch - tail, nch):
            pltpu.async_copy(src_hbm.at[idx_r.at[j]], a0, sga).wait()
            pltpu.async_copy(a0, acc.at[idx_c.at[j]], ssa, add=True).wait()

        plsc.subcore_barrier()
        pltpu.sync_copy(acc.at[sl], out_hbm.at[cid, sl])

    return agg_kernel


def _mm_body(x_ref, w_ref, h_ref):
    h_ref[...] = jnp.dot(x_ref[...], w_ref[...],
                         preferred_element_type=jnp.float32)


def _scale_body(h_ref, degp_ref, hs_ref, dinv_ref):
    deg = degp_ref[0, :, 0:1] + degp_ref[1, :, 0:1] + 1.0
    dinv = lax.rsqrt(deg)
    hs = dinv * h_ref[...]
    half = hs.shape[1] // 2
    hs_ref[0] = hs[:, :half]
    hs_ref[1] = hs[:, half:]
    dinv_ref[...] = jnp.broadcast_to(dinv, dinv_ref.shape)


def _mid_body(p_ref, hs_ref, dinv_ref, b1_ref, g_ref, be_ref, w2_ref,
              h2s_ref):
    dinv = dinv_ref[:, 0:1]
    s = jnp.concatenate([p_ref[0] + hs_ref[0], p_ref[1] + hs_ref[1]],
                        axis=-1)
    t = dinv * s + b1_ref[...]
    m = jnp.mean(t, axis=-1, keepdims=True)
    v = jnp.mean((t - m) ** 2, axis=-1, keepdims=True)
    t = (t - m) * lax.rsqrt(v + 1e-6) * g_ref[...] + be_ref[...]
    a = jnp.maximum(t, 0.0)
    h2 = jnp.dot(a, w2_ref[...], preferred_element_type=jnp.float32)
    h2s_ref[...] = dinv * h2


def _final_body(p_ref, h2s_ref, dinv_ref, b2_ref, out_ref):
    dinv = dinv_ref[:, 0:1]
    out_ref[...] = dinv * (p_ref[0] + p_ref[1] + h2s_ref[...]) + b2_ref[...]


def kernel(x, edge_index, edge_weight, W1, b1, gamma1, beta1, W2, b2):
    n0, d = x.shape
    hdim = W1.shape[1]
    ncls = W2.shape[1]
    e = edge_index.shape[1]
    n = ((n0 + NS * 8 - 1) // (NS * 8)) * (NS * 8)
    n = max(n, 10240)
    rows_per_tile = n // NS
    d2 = 64
    half = hdim // 2

    row = edge_index[0]
    col = edge_index[1]
    nch_e = e // (NW * K)
    row_e = row.reshape(NW, nch_e, K)
    col_e = col.reshape(NW, nch_e, K)
    nch_f = e // (NS * K)
    row_f = row.reshape(NS, nch_f, K)
    col_f = col.reshape(NS, nch_f, K)

    ones_deg = jnp.ones((K, DEG_W), jnp.float32)
    zeros_deg = jnp.zeros((rows_per_tile, DEG_W), jnp.float32)
    zeros_2 = jnp.zeros((rows_per_tile, d2), jnp.float32)
    W2p = jnp.pad(W2, ((0, 0), (0, d2 - ncls)))
    b2p = jnp.pad(b2, (0, d2 - ncls)).reshape(1, d2)

    R = 640
    grid = (n // R,)

    h1 = pl.pallas_call(
        _mm_body,
        grid=grid,
        in_specs=[
            pl.BlockSpec((R, d), lambda i: (i, 0)),
            pl.BlockSpec((d, hdim), lambda i: (0, 0)),
        ],
        out_specs=pl.BlockSpec((R, hdim), lambda i: (i, 0)),
        out_shape=jax.ShapeDtypeStruct((n, hdim), jnp.float32),
    )(x, W1)

    degp = _make_deg_kernel(n, e)(col_e, ones_deg, zeros_deg)

    hs12, dinv = pl.pallas_call(
        _scale_body,
        grid=grid,
        in_specs=[
            pl.BlockSpec((R, hdim), lambda i: (i, 0)),
            pl.BlockSpec((NC, R, DEG_W), lambda i: (0, i, 0)),
        ],
        out_specs=[
            pl.BlockSpec((NC, R, half), lambda i: (0, i, 0)),
            pl.BlockSpec((R, 8), lambda i: (i, 0)),
        ],
        out_shape=[
            jax.ShapeDtypeStruct((NC, n, half), jnp.float32),
            jax.ShapeDtypeStruct((n, 8), jnp.float32),
        ],
    )(h1, degp)

    p1 = _make_agg_kernel(n, e, half, True)(hs12, row_f, col_f, zeros_2)

    h2s = pl.pallas_call(
        _mid_body,
        grid=grid,
        in_specs=[
            pl.BlockSpec((NC, R, half), lambda i: (0, i, 0)),
            pl.BlockSpec((NC, R, half), lambda i: (0, i, 0)),
            pl.BlockSpec((R, 8), lambda i: (i, 0)),
            pl.BlockSpec((1, hdim), lambda i: (0, 0)),
            pl.BlockSpec((1, hdim), lambda i: (0, 0)),
            pl.BlockSpec((1, hdim), lambda i: (0, 0)),
            pl.BlockSpec((hdim, d2), lambda i: (0, 0)),
        ],
        out_specs=pl.BlockSpec((R, d2), lambda i: (i, 0)),
        out_shape=jax.ShapeDtypeStruct((n, d2), jnp.float32),
    )(p1, hs12, dinv, b1.reshape(1, hdim), gamma1.reshape(1, hdim),
      beta1.reshape(1, hdim), W2p)

    p2 = _make_agg_kernel(n, e, d2, False)(h2s, row_e, col_e, zeros_2)

    out = pl.pallas_call(
        _final_body,
        grid=grid,
        in_specs=[
            pl.BlockSpec((NC, R, d2), lambda i: (0, i, 0)),
            pl.BlockSpec((R, d2), lambda i: (i, 0)),
            pl.BlockSpec((R, 8), lambda i: (i, 0)),
            pl.BlockSpec((1, d2), lambda i: (0, 0)),
        ],
        out_specs=pl.BlockSpec((R, d2), lambda i: (i, 0)),
        out_shape=jax.ShapeDtypeStruct((n, d2), jnp.float32),
    )(p2, h2s, dinv, b2p)

    return out[:n0, :ncls]


# speedup vs baseline: 31.8029x; 1.0674x over previous
import functools

import jax
import jax.numpy as jnp
from jax import lax
from jax.experimental import pallas as pl
from jax.experimental.pallas import tpu as pltpu
from jax.experimental.pallas import tpu_sc as plsc

NC = 2
NS = 16
NW = NC * NS

K = 100
DEG_W = 8


def _sc_mesh():
    return plsc.VectorSubcoreMesh(
        core_axis_name="c", subcore_axis_name="s", num_cores=NC,
        num_subcores=NS)


def _make_deg_kernel(n, e):
    nch = e // (NW * K)
    rows_per_tile = n // NS

    @functools.partial(
        pl.kernel,
        out_type=jax.ShapeDtypeStruct((NC, n, DEG_W), jnp.float32),
        mesh=_sc_mesh(),
        compiler_params=pltpu.CompilerParams(use_tc_tiling_on_sc=False),
        scratch_types=[
            pltpu.VMEM((nch, K), jnp.int32),
            pltpu.VMEM((K, DEG_W), jnp.float32),
            pltpu.VMEM_SHARED((n, DEG_W), jnp.float32),
        ],
    )
    def deg_kernel(col_hbm, ones_hbm, zeros_hbm, out_hbm, idx_c, ones_v,
                   acc):
        cid = lax.axis_index("c")
        sid = lax.axis_index("s")
        wid = sid * NC + cid
        pltpu.sync_copy(col_hbm.at[wid], idx_c)
        pltpu.sync_copy(ones_hbm, ones_v)
        sl = pl.ds(sid * rows_per_tile, rows_per_tile)
        pltpu.sync_copy(zeros_hbm, acc.at[sl])
        plsc.subcore_barrier()

        def step(j, carry):
            pltpu.sync_copy(ones_v, acc.at[idx_c.at[j]], add=True)
            return carry

        lax.fori_loop(0, nch, step, 0)
        plsc.subcore_barrier()
        pltpu.sync_copy(acc.at[sl], out_hbm.at[cid, sl])

    return deg_kernel


def _make_agg_kernel(n, e, d, feature_split):
    tiles = NS if feature_split else NW
    nch = e // (tiles * K)
    rows_per_tile = n // NS

    @functools.partial(
        pl.kernel,
        out_type=jax.ShapeDtypeStruct((n, 2 * d), jnp.float32),
        mesh=_sc_mesh(),
        compiler_params=pltpu.CompilerParams(use_tc_tiling_on_sc=False),
        scratch_types=(
            [pltpu.VMEM((nch, K), jnp.int32)] * 2
            + [pltpu.VMEM((K, d), jnp.float32)] * 6
            + [pltpu.VMEM_SHARED((n, d), jnp.float32)]
            + [pltpu.SemaphoreType.DMA] * 4
        ),
    )
    def agg_kernel(h_hbm, row_hbm, col_hbm, zeros_hbm, out_hbm, idx_r,
                   idx_c, a0, a1, a2, b0, b1, b2, acc, sga, sgb,
                   ssa, ssb):
        cid = lax.axis_index("c")
        sid = lax.axis_index("s")
        src_hbm = h_hbm
        if feature_split:
            pltpu.sync_copy(row_hbm.at[cid, sid], idx_r)
            pltpu.sync_copy(col_hbm.at[sid], idx_c)
        else:
            wid = sid * NC + cid
            pltpu.sync_copy(row_hbm.at[wid], idx_r)
            pltpu.sync_copy(col_hbm.at[wid], idx_c)
        sl = pl.ds(sid * rows_per_tile, rows_per_tile)
        pltpu.sync_copy(zeros_hbm, acc.at[sl])
        plsc.subcore_barrier()

        bufa = (a0, a1, a2)
        bufb = (b0, b1, b2)
        G = 3

        def fire_g(base, bufs, sem):
            for t in range(G):
                pltpu.async_copy(src_hbm.at[idx_r.at[base + t]], bufs[t],
                                 sem)

        def drain_g(base, bufs, sem):
            for t in range(G):
                pltpu.make_async_copy(src_hbm.at[idx_r.at[base + t]],
                                      bufs[t], sem).wait()

        def fire_s(base, bufs, sem):
            for t in range(G):
                pltpu.async_copy(bufs[t], acc.at[idx_c.at[base + t]], sem,
                                 add=True)

        def drain_s(base, bufs, sem):
            for t in range(G):
                pltpu.make_async_copy(bufs[t], acc.at[idx_c.at[base + t]],
                                      sem).wait()

        niter = nch // (2 * G)
        tail = nch - niter * 2 * G

        if niter > 0:
            fire_g(0, bufa, sga)

            def step(i, carry):
                base = i * 2 * G
                drain_g(base, bufa, sga)

                @pl.when(i > 0)
                def _():
                    drain_s(base - G, bufb, ssb)

                fire_g(base + G, bufb, sgb)
                fire_s(base, bufa, ssa)
                drain_g(base + G, bufb, sgb)
                drain_s(base, bufa, ssa)

                @pl.when(i < niter - 1)
                def _():
                    fire_g(base + 2 * G, bufa, sga)

                fire_s(base + G, bufb, ssb)
                return carry

            lax.fori_loop(0, niter, step, 0)
            drain_s(niter * 2 * G - G, bufb, ssb)

        for j in range(nch - tail, nch):
            pltpu.async_copy(src_hbm.at[idx_r.at[j]], a0, sga).wait()
            pltpu.async_copy(a0, acc.at[idx_c.at[j]], ssa, add=True).wait()

        plsc.subcore_barrier()
        pltpu.sync_copy(acc.at[sl], out_hbm.at[sl, pl.ds(cid * d, d)])

    return agg_kernel


def _mm_body(x_ref, w_ref, h_ref):
    h_ref[...] = jnp.dot(x_ref[...], w_ref[...],
                         preferred_element_type=jnp.float32)


def _scale_body(h_ref, degp_ref, hs_ref, dinv_ref):
    deg = degp_ref[0, :, 0:1] + degp_ref[1, :, 0:1] + 1.0
    dinv = lax.rsqrt(deg)
    hs_ref[...] = dinv * h_ref[...]
    dinv_ref[...] = jnp.broadcast_to(dinv, dinv_ref.shape)


def _mid_body(p_ref, hs_ref, dinv_ref, b1_ref, g_ref, be_ref, w2_ref,
              h2s_ref):
    dinv = dinv_ref[:, 0:1]
    t = dinv * (p_ref[...] + hs_ref[...]) + b1_ref[...]
    m = jnp.mean(t, axis=-1, keepdims=True)
    v = jnp.mean((t - m) ** 2, axis=-1, keepdims=True)
    t = (t - m) * lax.rsqrt(v + 1e-6) * g_ref[...] + be_ref[...]
    a = jnp.maximum(t, 0.0)
    h2 = jnp.dot(a, w2_ref[...], preferred_element_type=jnp.float32)
    s2 = dinv * h2
    h2s_ref[...] = jnp.concatenate([s2, s2], axis=-1)


def _final_body(p_ref, h2s_ref, dinv_ref, b2_ref, out_ref):
    dinv = dinv_ref[:, 0:1]
    d = out_ref.shape[-1]
    psum = p_ref[:, :d] + p_ref[:, d:]
    out_ref[...] = dinv * (psum + h2s_ref[:, :d]) + b2_ref[...]


def kernel(x, edge_index, edge_weight, W1, b1, gamma1, beta1, W2, b2):
    n0, d = x.shape
    hdim = W1.shape[1]
    ncls = W2.shape[1]
    e = edge_index.shape[1]
    n = ((n0 + NS * 8 - 1) // (NS * 8)) * (NS * 8)
    n = max(n, 10240)
    rows_per_tile = n // NS
    d2 = 64
    half = hdim // 2

    row = edge_index[0]
    col = edge_index[1]
    row2 = row * 2
    nch_e = e // (NW * K)
    row_e2 = row2.reshape(NW, nch_e, K)
    col_e = col.reshape(NW, nch_e, K)
    nch_f = e // (NS * K)
    row_f2 = jnp.stack([row2, row2 + 1]).reshape(NC, NS, nch_f, K)
    col_f = col.reshape(NS, nch_f, K)

    ones_deg = jnp.ones((K, DEG_W), jnp.float32)
    zeros_deg = jnp.zeros((rows_per_tile, DEG_W), jnp.float32)
    zeros_2 = jnp.zeros((rows_per_tile, d2), jnp.float32)
    W2p = jnp.pad(W2, ((0, 0), (0, d2 - ncls)))
    b2p = jnp.pad(b2, (0, d2 - ncls)).reshape(1, d2)

    R = 640
    grid = (n // R,)

    h1 = pl.pallas_call(
        _mm_body,
        grid=grid,
        in_specs=[
            pl.BlockSpec((R, d), lambda i: (i, 0)),
            pl.BlockSpec((d, hdim), lambda i: (0, 0)),
        ],
        out_specs=pl.BlockSpec((R, hdim), lambda i: (i, 0)),
        out_shape=jax.ShapeDtypeStruct((n, hdim), jnp.float32),
    )(x, W1)

    degp = _make_deg_kernel(n, e)(col_e, ones_deg, zeros_deg)

    hs, dinv = pl.pallas_call(
        _scale_body,
        grid=grid,
        in_specs=[
            pl.BlockSpec((R, hdim), lambda i: (i, 0)),
            pl.BlockSpec((NC, R, DEG_W), lambda i: (0, i, 0)),
        ],
        out_specs=[
            pl.BlockSpec((R, hdim), lambda i: (i, 0)),
            pl.BlockSpec((R, 8), lambda i: (i, 0)),
        ],
        out_shape=[
            jax.ShapeDtypeStruct((n, hdim), jnp.float32),
            jax.ShapeDtypeStruct((n, 8), jnp.float32),
        ],
    )(h1, degp)

    p1 = _make_agg_kernel(n, e, half, True)(
        hs.reshape(2 * n, half), row_f2, col_f, zeros_2)

    h2s = pl.pallas_call(
        _mid_body,
        grid=grid,
        in_specs=[
            pl.BlockSpec((R, hdim), lambda i: (i, 0)),
            pl.BlockSpec((R, hdim), lambda i: (i, 0)),
            pl.BlockSpec((R, 8), lambda i: (i, 0)),
            pl.BlockSpec((1, hdim), lambda i: (0, 0)),
            pl.BlockSpec((1, hdim), lambda i: (0, 0)),
            pl.BlockSpec((1, hdim), lambda i: (0, 0)),
            pl.BlockSpec((hdim, d2), lambda i: (0, 0)),
        ],
        out_specs=pl.BlockSpec((R, 2 * d2), lambda i: (i, 0)),
        out_shape=jax.ShapeDtypeStruct((n, 2 * d2), jnp.float32),
    )(p1, hs, dinv, b1.reshape(1, hdim), gamma1.reshape(1, hdim),
      beta1.reshape(1, hdim), W2p)

    p2 = _make_agg_kernel(n, e, d2, False)(
        h2s.reshape(2 * n, d2), row_e2, col_e, zeros_2)

    out = pl.pallas_call(
        _final_body,
        grid=grid,
        in_specs=[
            pl.BlockSpec((R, 2 * d2), lambda i: (i, 0)),
            pl.BlockSpec((R, 2 * d2), lambda i: (i, 0)),
            pl.BlockSpec((R, 8), lambda i: (i, 0)),
            pl.BlockSpec((1, d2), lambda i: (0, 0)),
        ],
        out_specs=pl.BlockSpec((R, d2), lambda i: (i, 0)),
        out_shape=jax.ShapeDtypeStruct((n, d2), jnp.float32),
    )(p2, h2s, dinv, b2p)

    return out[:n0, :ncls]


# speedup vs baseline: 33.0222x; 1.0383x over previous
import functools

import jax
import jax.numpy as jnp
from jax import lax
from jax.experimental import pallas as pl
from jax.experimental.pallas import tpu as pltpu
from jax.experimental.pallas import tpu_sc as plsc

NC = 2
NS = 16
NW = NC * NS

K = 100
DEG_W = 8


def _sc_mesh():
    return plsc.VectorSubcoreMesh(
        core_axis_name="c", subcore_axis_name="s", num_cores=NC,
        num_subcores=NS)


def _make_deg_kernel(n, e):
    nch = e // (NW * K)
    rows_per_tile = n // NS

    @functools.partial(
        pl.kernel,
        out_type=jax.ShapeDtypeStruct((NC, n, DEG_W), jnp.float32),
        mesh=_sc_mesh(),
        compiler_params=pltpu.CompilerParams(use_tc_tiling_on_sc=False),
        scratch_types=[
            pltpu.VMEM((nch, K), jnp.int32),
            pltpu.VMEM((K, DEG_W), jnp.float32),
            pltpu.VMEM_SHARED((n, DEG_W), jnp.float32),
        ],
    )
    def deg_kernel(col_hbm, ones_hbm, zeros_hbm, out_hbm, idx_c, ones_v,
                   acc):
        cid = lax.axis_index("c")
        sid = lax.axis_index("s")
        wid = sid * NC + cid
        pltpu.sync_copy(col_hbm.at[wid], idx_c)
        pltpu.sync_copy(ones_hbm, ones_v)
        sl = pl.ds(sid * rows_per_tile, rows_per_tile)
        pltpu.sync_copy(zeros_hbm, acc.at[sl])
        plsc.subcore_barrier()

        def step(j, carry):
            pltpu.sync_copy(ones_v, acc.at[idx_c.at[j]], add=True)
            return carry

        lax.fori_loop(0, nch, step, 0)
        plsc.subcore_barrier()
        pltpu.sync_copy(acc.at[sl], out_hbm.at[cid, sl])

    return deg_kernel


def _make_agg_kernel(n, e, d, feature_split):
    tiles = NS if feature_split else NW
    nch = e // (tiles * K)
    rows_per_tile = n // NS

    @functools.partial(
        pl.kernel,
        out_type=jax.ShapeDtypeStruct((n, 2 * d), jnp.float32),
        mesh=_sc_mesh(),
        compiler_params=pltpu.CompilerParams(use_tc_tiling_on_sc=False),
        scratch_types=(
            [pltpu.VMEM((nch, K), jnp.int32)] * 2
            + [pltpu.VMEM((K, d), jnp.float32)] * 6
            + [pltpu.VMEM_SHARED((n, d), jnp.float32)]
            + [pltpu.SemaphoreType.DMA] * 4
        ),
    )
    def agg_kernel(h_hbm, row_hbm, col_hbm, zeros_hbm, out_hbm, idx_r,
                   idx_c, a0, a1, a2, b0, b1, b2, acc, sga, sgb,
                   ssa, ssb):
        cid = lax.axis_index("c")
        sid = lax.axis_index("s")
        if feature_split:
            src_hbm = h_hbm.at[pl.ds(cid, h_hbm.shape[0] - 1)]
            pltpu.sync_copy(row_hbm.at[sid], idx_r)
            pltpu.sync_copy(col_hbm.at[sid], idx_c)
        else:
            src_hbm = h_hbm.at[pl.ds(0, h_hbm.shape[0] - 1)]
            wid = sid * NC + cid
            pltpu.sync_copy(row_hbm.at[wid], idx_r)
            pltpu.sync_copy(col_hbm.at[wid], idx_c)
        sl = pl.ds(sid * rows_per_tile, rows_per_tile)
        pltpu.sync_copy(zeros_hbm, acc.at[sl])
        plsc.subcore_barrier()

        bufa = (a0, a1, a2)
        bufb = (b0, b1, b2)
        G = 3

        def fire_g(base, bufs, sem):
            for t in range(G):
                pltpu.async_copy(src_hbm.at[idx_r.at[base + t]], bufs[t],
                                 sem)

        def drain_g(base, bufs, sem):
            for t in range(G):
                pltpu.make_async_copy(src_hbm.at[idx_r.at[base + t]],
                                      bufs[t], sem).wait()

        def fire_s(base, bufs, sem):
            for t in range(G):
                pltpu.async_copy(bufs[t], acc.at[idx_c.at[base + t]], sem,
                                 add=True)

        def drain_s(base, bufs, sem):
            for t in range(G):
                pltpu.make_async_copy(bufs[t], acc.at[idx_c.at[base + t]],
                                      sem).wait()

        niter = nch // (2 * G)
        tail = nch - niter * 2 * G

        if niter > 0:
            fire_g(0, bufa, sga)

            def step(i, carry):
                base = i * 2 * G
                drain_g(base, bufa, sga)

                @pl.when(i > 0)
                def _():
                    drain_s(base - G, bufb, ssb)

                fire_g(base + G, bufb, sgb)
                fire_s(base, bufa, ssa)
                drain_g(base + G, bufb, sgb)
                drain_s(base, bufa, ssa)

                @pl.when(i < niter - 1)
                def _():
                    fire_g(base + 2 * G, bufa, sga)

                fire_s(base + G, bufb, ssb)
                return carry

            lax.fori_loop(0, niter, step, 0)
            drain_s(niter * 2 * G - G, bufb, ssb)

        for j in range(nch - tail, nch):
            pltpu.async_copy(src_hbm.at[idx_r.at[j]], a0, sga).wait()
            pltpu.async_copy(a0, acc.at[idx_c.at[j]], ssa, add=True).wait()

        plsc.subcore_barrier()
        pltpu.sync_copy(acc.at[sl], out_hbm.at[sl, pl.ds(cid * d, d)])

    return agg_kernel


def _mm_body(x_ref, w_ref, h_ref):
    h_ref[...] = jnp.dot(x_ref[...], w_ref[...],
                         preferred_element_type=jnp.float32)


def _scale_body(h_ref, degp_ref, hs_ref, dinv_ref):
    deg = degp_ref[0, :, 0:1] + degp_ref[1, :, 0:1] + 1.0
    dinv = lax.rsqrt(deg)
    hs_ref[...] = dinv * h_ref[...]
    dinv_ref[...] = jnp.broadcast_to(dinv, dinv_ref.shape)


def _mid_body(p_ref, hs_ref, dinv_ref, b1_ref, g_ref, be_ref, w2_ref,
              h2s_ref):
    dinv = dinv_ref[:, 0:1]
    t = dinv * (p_ref[...] + hs_ref[...]) + b1_ref[...]
    m = jnp.mean(t, axis=-1, keepdims=True)
    v = jnp.mean((t - m) ** 2, axis=-1, keepdims=True)
    t = (t - m) * lax.rsqrt(v + 1e-6) * g_ref[...] + be_ref[...]
    a = jnp.maximum(t, 0.0)
    h2 = jnp.dot(a, w2_ref[...], preferred_element_type=jnp.float32)
    s2 = dinv * h2
    h2s_ref[...] = jnp.concatenate([s2, s2], axis=-1)


def _final_body(p_ref, h2s_ref, dinv_ref, b2_ref, out_ref):
    dinv = dinv_ref[:, 0:1]
    d = out_ref.shape[-1]
    psum = p_ref[:, :d] + p_ref[:, d:]
    out_ref[...] = dinv * (psum + h2s_ref[:, :d]) + b2_ref[...]


def kernel(x, edge_index, edge_weight, W1, b1, gamma1, beta1, W2, b2):
    n0, d = x.shape
    hdim = W1.shape[1]
    ncls = W2.shape[1]
    e = edge_index.shape[1]
    n = ((n0 + NS * 8 - 1) // (NS * 8)) * (NS * 8)
    n = max(n, 10240)
    rows_per_tile = n // NS
    d2 = 64
    half = hdim // 2

    row = edge_index[0]
    col = edge_index[1]
    row2 = row * 2
    nch_e = e // (NW * K)
    row_e2 = row2.reshape(NW, nch_e, K)
    col_e = col.reshape(NW, nch_e, K)
    nch_f = e // (NS * K)
    row_f2 = row2.reshape(NS, nch_f, K)
    col_f = col.reshape(NS, nch_f, K)

    ones_deg = jnp.ones((K, DEG_W), jnp.float32)
    zeros_deg = jnp.zeros((rows_per_tile, DEG_W), jnp.float32)
    zeros_2 = jnp.zeros((rows_per_tile, d2), jnp.float32)
    W2p = jnp.pad(W2, ((0, 0), (0, d2 - ncls)))
    b2p = jnp.pad(b2, (0, d2 - ncls)).reshape(1, d2)

    R = 640
    grid = (n // R,)

    h1 = pl.pallas_call(
        _mm_body,
        grid=grid,
        in_specs=[
            pl.BlockSpec((R, d), lambda i: (i, 0)),
            pl.BlockSpec((d, hdim), lambda i: (0, 0)),
        ],
        out_specs=pl.BlockSpec((R, hdim), lambda i: (i, 0)),
        out_shape=jax.ShapeDtypeStruct((n, hdim), jnp.float32),
    )(x, W1)

    degp = _make_deg_kernel(n, e)(col_e, ones_deg, zeros_deg)

    hs, dinv = pl.pallas_call(
        _scale_body,
        grid=grid,
        in_specs=[
            pl.BlockSpec((R, hdim), lambda i: (i, 0)),
            pl.BlockSpec((NC, R, DEG_W), lambda i: (0, i, 0)),
        ],
        out_specs=[
            pl.BlockSpec((R, hdim), lambda i: (i, 0)),
            pl.BlockSpec((R, 8), lambda i: (i, 0)),
        ],
        out_shape=[
            jax.ShapeDtypeStruct((n, hdim), jnp.float32),
            jax.ShapeDtypeStruct((n, 8), jnp.float32),
        ],
    )(h1, degp)

    p1 = _make_agg_kernel(n, e, half, True)(
        hs.reshape(2 * n, half), row_f2, col_f, zeros_2)

    h2s = pl.pallas_call(
        _mid_body,
        grid=grid,
        in_specs=[
            pl.BlockSpec((R, hdim), lambda i: (i, 0)),
            pl.BlockSpec((R, hdim), lambda i: (i, 0)),
            pl.BlockSpec((R, 8), lambda i: (i, 0)),
            pl.BlockSpec((1, hdim), lambda i: (0, 0)),
            pl.BlockSpec((1, hdim), lambda i: (0, 0)),
            pl.BlockSpec((1, hdim), lambda i: (0, 0)),
            pl.BlockSpec((hdim, d2), lambda i: (0, 0)),
        ],
        out_specs=pl.BlockSpec((R, 2 * d2), lambda i: (i, 0)),
        out_shape=jax.ShapeDtypeStruct((n, 2 * d2), jnp.float32),
    )(p1, hs, dinv, b1.reshape(1, hdim), gamma1.reshape(1, hdim),
      beta1.reshape(1, hdim), W2p)

    p2 = _make_agg_kernel(n, e, d2, False)(
        h2s.reshape(2 * n, d2), row_e2, col_e, zeros_2)

    out = pl.pallas_call(
        _final_body,
        grid=grid,
        in_specs=[
            pl.BlockSpec((R, 2 * d2), lambda i: (i, 0)),
            pl.BlockSpec((R, 2 * d2), lambda i: (i, 0)),
            pl.BlockSpec((R, 8), lambda i: (i, 0)),
            pl.BlockSpec((1, d2), lambda i: (0, 0)),
        ],
        out_specs=pl.BlockSpec((R, d2), lambda i: (i, 0)),
        out_shape=jax.ShapeDtypeStruct((n, d2), jnp.float32),
    )(p2, h2s, dinv, b2p)

    return out[:n0, :ncls]


# speedup vs baseline: 34.3158x; 1.0392x over previous
import functools

import jax
import jax.numpy as jnp
from jax import lax
from jax.experimental import pallas as pl
from jax.experimental.pallas import tpu as pltpu
from jax.experimental.pallas import tpu_sc as plsc

NC = 2
NS = 16
NW = NC * NS

K = 100
DEG_W = 8


def _sc_mesh():
    return plsc.VectorSubcoreMesh(
        core_axis_name="c", subcore_axis_name="s", num_cores=NC,
        num_subcores=NS)


def _make_deg_kernel(n, e):
    nch = e // (NW * K)
    rows_per_tile = n // NS

    @functools.partial(
        pl.kernel,
        out_type=jax.ShapeDtypeStruct((NC, n, DEG_W), jnp.float32),
        mesh=_sc_mesh(),
        compiler_params=pltpu.CompilerParams(use_tc_tiling_on_sc=False),
        scratch_types=[
            pltpu.VMEM((nch, K), jnp.int32),
            pltpu.VMEM((K, DEG_W), jnp.float32),
            pltpu.VMEM_SHARED((n, DEG_W), jnp.float32),
        ],
    )
    def deg_kernel(col_hbm, ones_hbm, zeros_hbm, out_hbm, idx_c, ones_v,
                   acc):
        cid = lax.axis_index("c")
        sid = lax.axis_index("s")
        wid = sid * NC + cid
        pltpu.sync_copy(col_hbm.at[wid], idx_c)
        pltpu.sync_copy(ones_hbm, ones_v)
        sl = pl.ds(sid * rows_per_tile, rows_per_tile)
        pltpu.sync_copy(zeros_hbm, acc.at[sl])
        plsc.subcore_barrier()

        def step(j, carry):
            pltpu.sync_copy(ones_v, acc.at[idx_c.at[j]], add=True)
            return carry

        lax.fori_loop(0, nch, step, 0)
        plsc.subcore_barrier()
        pltpu.sync_copy(acc.at[sl], out_hbm.at[cid, sl])

    return deg_kernel


def _make_agg_kernel(n, e, d, feature_split):
    tiles = NS if feature_split else NW
    nch = e // (tiles * K)
    rows_per_tile = n // NS

    @functools.partial(
        pl.kernel,
        out_type=jax.ShapeDtypeStruct((n, 2 * d), jnp.float32),
        mesh=_sc_mesh(),
        compiler_params=pltpu.CompilerParams(use_tc_tiling_on_sc=False),
        scratch_types=(
            [pltpu.VMEM((nch, K), jnp.int32)] * 2
            + [pltpu.VMEM((K, d), jnp.float32)] * 6
            + [pltpu.VMEM_SHARED((n, d), jnp.float32)]
            + [pltpu.SemaphoreType.DMA] * 4
        ),
    )
    def agg_kernel(h_hbm, row_hbm, col_hbm, zeros_hbm, out_hbm, idx_r,
                   idx_c, a0, a1, a2, b0, b1, b2, acc, sga, sgb,
                   ssa, ssb):
        cid = lax.axis_index("c")
        sid = lax.axis_index("s")
        if feature_split:
            src_hbm = h_hbm.at[pl.ds(cid, h_hbm.shape[0] - 1)]
            pltpu.sync_copy(row_hbm.at[sid], idx_r)
            pltpu.sync_copy(col_hbm.at[sid], idx_c)
        else:
            src_hbm = h_hbm.at[pl.ds(0, h_hbm.shape[0] - 1)]
            wid = sid * NC + cid
            pltpu.sync_copy(row_hbm.at[wid], idx_r)
            pltpu.sync_copy(col_hbm.at[wid], idx_c)
        sl = pl.ds(sid * rows_per_tile, rows_per_tile)
        pltpu.sync_copy(zeros_hbm, acc.at[sl])
        plsc.subcore_barrier()

        bufa = (a0, a1, a2)
        bufb = (b0, b1, b2)
        G = 3

        def fire_g(base, bufs, sem):
            for t in range(G):
                pltpu.async_copy(src_hbm.at[idx_r.at[base + t]], bufs[t],
                                 sem)

        def drain_g(base, bufs, sem):
            for t in range(G):
                pltpu.make_async_copy(src_hbm.at[idx_r.at[base + t]],
                                      bufs[t], sem).wait()

        def fire_s(base, bufs, sem):
            for t in range(G):
                pltpu.async_copy(bufs[t], acc.at[idx_c.at[base + t]], sem,
                                 add=True)

        def drain_s(base, bufs, sem):
            for t in range(G):
                pltpu.make_async_copy(bufs[t], acc.at[idx_c.at[base + t]],
                                      sem).wait()

        niter = nch // (2 * G)
        tail = nch - niter * 2 * G

        if niter > 0:
            fire_g(0, bufa, sga)

            def step(i, carry):
                base = i * 2 * G
                drain_g(base, bufa, sga)

                @pl.when(i > 0)
                def _():
                    drain_s(base - G, bufb, ssb)

                fire_g(base + G, bufb, sgb)
                fire_s(base, bufa, ssa)
                drain_g(base + G, bufb, sgb)
                drain_s(base, bufa, ssa)

                @pl.when(i < niter - 1)
                def _():
                    fire_g(base + 2 * G, bufa, sga)

                fire_s(base + G, bufb, ssb)
                return carry

            lax.fori_loop(0, niter, step, 0)
            drain_s(niter * 2 * G - G, bufb, ssb)

        for j in range(nch - tail, nch):
            pltpu.async_copy(src_hbm.at[idx_r.at[j]], a0, sga).wait()
            pltpu.async_copy(a0, acc.at[idx_c.at[j]], ssa, add=True).wait()

        plsc.subcore_barrier()
        pltpu.sync_copy(acc.at[sl], out_hbm.at[sl, pl.ds(cid * d, d)])

    return agg_kernel


def _mm1_body(x_ref, w_ref, degp_ref, hs_ref, dinv_ref):
    deg = degp_ref[0, :, 0:1] + degp_ref[1, :, 0:1] + 1.0
    dinv = lax.rsqrt(deg)
    h = jnp.dot(x_ref[...], w_ref[...], preferred_element_type=jnp.float32)
    hs_ref[...] = dinv * h
    dinv_ref[...] = jnp.broadcast_to(dinv, dinv_ref.shape)


def _mid_body(p_ref, hs_ref, dinv_ref, b1_ref, g_ref, be_ref, w2_ref,
              h2s_ref):
    dinv = dinv_ref[:, 0:1]
    t = dinv * (p_ref[...] + hs_ref[...]) + b1_ref[...]
    m = jnp.mean(t, axis=-1, keepdims=True)
    v = jnp.mean((t - m) ** 2, axis=-1, keepdims=True)
    t = (t - m) * lax.rsqrt(v + 1e-6) * g_ref[...] + be_ref[...]
    a = jnp.maximum(t, 0.0)
    h2 = jnp.dot(a, w2_ref[...], preferred_element_type=jnp.float32)
    s2 = dinv * h2
    h2s_ref[...] = jnp.concatenate([s2, s2], axis=-1)


def _final_body(p_ref, h2s_ref, dinv_ref, b2_ref, out_ref):
    dinv = dinv_ref[:, 0:1]
    d = p_ref.shape[-1] // 2
    ncls = out_ref.shape[-1]
    psum = p_ref[:, :d] + p_ref[:, d:]
    t = dinv * (psum + h2s_ref[:, :d])
    out_ref[...] = t[:, :ncls] + b2_ref[...]


def kernel(x, edge_index, edge_weight, W1, b1, gamma1, beta1, W2, b2):
    n0, d = x.shape
    hdim = W1.shape[1]
    ncls = W2.shape[1]
    e = edge_index.shape[1]
    n = ((n0 + NS * 8 - 1) // (NS * 8)) * (NS * 8)
    n = max(n, 10240)
    rows_per_tile = n // NS
    d2 = 64
    half = hdim // 2

    row = edge_index[0]
    col = edge_index[1]
    row2 = row * 2
    nch_e = e // (NW * K)
    row_e2 = row2.reshape(NW, nch_e, K)
    col_e = col.reshape(NW, nch_e, K)
    nch_f = e // (NS * K)
    row_f2 = row2.reshape(NS, nch_f, K)
    col_f = col.reshape(NS, nch_f, K)

    ones_deg = jnp.ones((K, DEG_W), jnp.float32)
    zeros_deg = jnp.zeros((rows_per_tile, DEG_W), jnp.float32)
    zeros_2 = jnp.zeros((rows_per_tile, d2), jnp.float32)
    W2p = jnp.pad(W2, ((0, 0), (0, d2 - ncls)))

    R = 640
    grid = (n // R,)

    degp = _make_deg_kernel(n, e)(col_e, ones_deg, zeros_deg)

    hs, dinv = pl.pallas_call(
        _mm1_body,
        grid=grid,
        in_specs=[
            pl.BlockSpec((R, d), lambda i: (i, 0)),
            pl.BlockSpec((d, hdim), lambda i: (0, 0)),
            pl.BlockSpec((NC, R, DEG_W), lambda i: (0, i, 0)),
        ],
        out_specs=[
            pl.BlockSpec((R, hdim), lambda i: (i, 0)),
            pl.BlockSpec((R, 8), lambda i: (i, 0)),
        ],
        out_shape=[
            jax.ShapeDtypeStruct((n, hdim), jnp.float32),
            jax.ShapeDtypeStruct((n, 8), jnp.float32),
        ],
    )(x, W1, degp)

    p1 = _make_agg_kernel(n, e, half, True)(
        hs.reshape(2 * n, half), row_f2, col_f, zeros_2)

    h2s = pl.pallas_call(
        _mid_body,
        grid=grid,
        in_specs=[
            pl.BlockSpec((R, hdim), lambda i: (i, 0)),
            pl.BlockSpec((R, hdim), lambda i: (i, 0)),
            pl.BlockSpec((R, 8), lambda i: (i, 0)),
            pl.BlockSpec((1, hdim), lambda i: (0, 0)),
            pl.BlockSpec((1, hdim), lambda i: (0, 0)),
            pl.BlockSpec((1, hdim), lambda i: (0, 0)),
            pl.BlockSpec((hdim, d2), lambda i: (0, 0)),
        ],
        out_specs=pl.BlockSpec((R, 2 * d2), lambda i: (i, 0)),
        out_shape=jax.ShapeDtypeStruct((n, 2 * d2), jnp.float32),
    )(p1, hs, dinv, b1.reshape(1, hdim), gamma1.reshape(1, hdim),
      beta1.reshape(1, hdim), W2p)

    p2 = _make_agg_kernel(n, e, d2, False)(
        h2s.reshape(2 * n, d2), row_e2, col_e, zeros_2)

    Rf = 1000
    out = pl.pallas_call(
        _final_body,
        grid=(n0 // Rf,),
        in_specs=[
            pl.BlockSpec((Rf, 2 * d2), lambda i: (i, 0)),
            pl.BlockSpec((Rf, 2 * d2), lambda i: (i, 0)),
            pl.BlockSpec((Rf, 8), lambda i: (i, 0)),
            pl.BlockSpec((1, ncls), lambda i: (0, 0)),
        ],
        out_specs=pl.BlockSpec((Rf, ncls), lambda i: (i, 0)),
        out_shape=jax.ShapeDtypeStruct((n0, ncls), jnp.float32),
    )(p2, h2s, dinv, b2.reshape(1, ncls))

    return out


# speedup vs baseline: 35.3687x; 1.0307x over previous
import functools

import jax
import jax.numpy as jnp
from jax import lax
from jax.experimental import pallas as pl
from jax.experimental.pallas import tpu as pltpu
from jax.experimental.pallas import tpu_sc as plsc

NC = 2
NS = 16
NW = NC * NS

K = 100
DEG_W = 8


def _sc_mesh():
    return plsc.VectorSubcoreMesh(
        core_axis_name="c", subcore_axis_name="s", num_cores=NC,
        num_subcores=NS)


def _make_deg_kernel(n, e):
    nch = e // (NW * K)
    rows_per_tile = n // NS

    @functools.partial(
        pl.kernel,
        out_type=jax.ShapeDtypeStruct((NC, n, DEG_W), jnp.float32),
        mesh=_sc_mesh(),
        compiler_params=pltpu.CompilerParams(use_tc_tiling_on_sc=False),
        scratch_types=[
            pltpu.VMEM((nch, K), jnp.int32),
            pltpu.VMEM((K, DEG_W), jnp.float32),
            pltpu.VMEM_SHARED((n, DEG_W), jnp.float32),
        ],
    )
    def deg_kernel(col_hbm, ones_hbm, zeros_hbm, out_hbm, idx_c, ones_v,
                   acc):
        cid = lax.axis_index("c")
        sid = lax.axis_index("s")
        wid = sid * NC + cid
        pltpu.sync_copy(col_hbm.at[wid], idx_c)
        pltpu.sync_copy(ones_hbm, ones_v)
        sl = pl.ds(sid * rows_per_tile, rows_per_tile)
        pltpu.sync_copy(zeros_hbm, acc.at[sl])
        plsc.subcore_barrier()

        def step(j, carry):
            pltpu.sync_copy(ones_v, acc.at[idx_c.at[j]], add=True)
            return carry

        lax.fori_loop(0, nch, step, 0)
        plsc.subcore_barrier()
        pltpu.sync_copy(acc.at[sl], out_hbm.at[cid, sl])

    return deg_kernel


def _make_agg_kernel(n, e, d, feature_split):
    tiles = NS if feature_split else NW
    nch = e // (tiles * K)
    rows_per_tile = n // NS

    @functools.partial(
        pl.kernel,
        out_type=jax.ShapeDtypeStruct((n, 2 * d), jnp.float32),
        mesh=_sc_mesh(),
        compiler_params=pltpu.CompilerParams(use_tc_tiling_on_sc=False),
        scratch_types=(
            [pltpu.VMEM((nch // 2, K), jnp.int32)] * 2
            + [pltpu.VMEM((K, d), jnp.float32)] * 10
            + [pltpu.VMEM_SHARED((n, d), jnp.float32)]
            + [pltpu.SemaphoreType.DMA] * 4
        ),
    )
    def agg_kernel(h_hbm, row_hbm, col_hbm, zeros_hbm, out_hbm, idx_r,
                   idx_c, a0, a1, a2, a3, a4, b0, b1, b2, b3, b4, acc,
                   sga, sgb, ssa, ssb):
        cid = lax.axis_index("c")
        sid = lax.axis_index("s")
        if feature_split:
            src_hbm = h_hbm.at[pl.ds(cid, h_hbm.shape[0] - 1)]
            tid = sid
        else:
            src_hbm = h_hbm.at[pl.ds(0, h_hbm.shape[0] - 1)]
            tid = sid * NC + cid
        rows_slab = row_hbm.at[tid]
        cols_slab = col_hbm.at[tid]
        sl = pl.ds(sid * rows_per_tile, rows_per_tile)
        pltpu.sync_copy(zeros_hbm, acc.at[sl])
        plsc.subcore_barrier()

        bufa = (a0, a1, a2, a3, a4)
        bufb = (b0, b1, b2, b3, b4)
        G = 5
        nch2 = nch // 2

        def fire_g(base, bufs, sem):
            for t in range(G):
                pltpu.async_copy(src_hbm.at[idx_r.at[base + t]], bufs[t],
                                 sem)

        def drain_g(base, bufs, sem):
            for t in range(G):
                pltpu.make_async_copy(src_hbm.at[idx_r.at[base + t]],
                                      bufs[t], sem).wait()

        def fire_s(base, bufs, sem):
            for t in range(G):
                pltpu.async_copy(bufs[t], acc.at[idx_c.at[base + t]], sem,
                                 add=True)

        def drain_s(base, bufs, sem):
            for t in range(G):
                pltpu.make_async_copy(bufs[t], acc.at[idx_c.at[base + t]],
                                      sem).wait()

        niter = nch2 // (2 * G)
        for ph in range(2):
            pltpu.sync_copy(rows_slab.at[pl.ds(ph * nch2, nch2)], idx_r)
            pltpu.sync_copy(cols_slab.at[pl.ds(ph * nch2, nch2)], idx_c)
            fire_g(0, bufa, sga)

            def step(i, carry):
                base = i * 2 * G
                drain_g(base, bufa, sga)

                @pl.when(i > 0)
                def _():
                    drain_s(base - G, bufb, ssb)

                fire_g(base + G, bufb, sgb)
                fire_s(base, bufa, ssa)
                drain_g(base + G, bufb, sgb)
                drain_s(base, bufa, ssa)

                @pl.when(i < niter - 1)
                def _():
                    fire_g(base + 2 * G, bufa, sga)

                fire_s(base + G, bufb, ssb)
                return carry

            lax.fori_loop(0, niter, step, 0)
            drain_s(niter * 2 * G - G, bufb, ssb)

        plsc.subcore_barrier()
        pltpu.sync_copy(acc.at[sl], out_hbm.at[sl, pl.ds(cid * d, d)])

    return agg_kernel


def _mm1_body(x_ref, w_ref, degp_ref, hs_ref, dinv_ref):
    deg = degp_ref[0, :, 0:1] + degp_ref[1, :, 0:1] + 1.0
    dinv = lax.rsqrt(deg)
    h = jnp.dot(x_ref[...], w_ref[...], preferred_element_type=jnp.float32)
    hs_ref[...] = dinv * h
    dinv_ref[...] = jnp.broadcast_to(dinv, dinv_ref.shape)


def _mid_body(p_ref, hs_ref, dinv_ref, b1_ref, g_ref, be_ref, w2_ref,
              h2s_ref):
    dinv = dinv_ref[:, 0:1]
    t = dinv * (p_ref[...] + hs_ref[...]) + b1_ref[...]
    m = jnp.mean(t, axis=-1, keepdims=True)
    v = jnp.mean((t - m) ** 2, axis=-1, keepdims=True)
    t = (t - m) * lax.rsqrt(v + 1e-6) * g_ref[...] + be_ref[...]
    a = jnp.maximum(t, 0.0)
    h2 = jnp.dot(a, w2_ref[...], preferred_element_type=jnp.float32)
    s2 = dinv * h2
    h2s_ref[...] = jnp.concatenate([s2, s2], axis=-1)


def _final_body(p_ref, h2s_ref, dinv_ref, b2_ref, out_ref):
    dinv = dinv_ref[:, 0:1]
    d = p_ref.shape[-1] // 2
    ncls = out_ref.shape[-1]
    psum = p_ref[:, :d] + p_ref[:, d:]
    t = dinv * (psum + h2s_ref[:, :d])
    out_ref[...] = t[:, :ncls] + b2_ref[...]


def kernel(x, edge_index, edge_weight, W1, b1, gamma1, beta1, W2, b2):
    n0, d = x.shape
    hdim = W1.shape[1]
    ncls = W2.shape[1]
    e = edge_index.shape[1]
    n = ((n0 + NS * 8 - 1) // (NS * 8)) * (NS * 8)
    n = max(n, 10240)
    rows_per_tile = n // NS
    d2 = 64
    half = hdim // 2

    row = edge_index[0]
    col = edge_index[1]
    row2 = row * 2
    nch_e = e // (NW * K)
    row_e2 = row2.reshape(NW, nch_e, K)
    col_e = col.reshape(NW, nch_e, K)
    nch_f = e // (NS * K)
    row_f2 = row2.reshape(NS, nch_f, K)
    col_f = col.reshape(NS, nch_f, K)

    ones_deg = jnp.ones((K, DEG_W), jnp.float32)
    zeros_deg = jnp.zeros((rows_per_tile, DEG_W), jnp.float32)
    zeros_2 = jnp.zeros((rows_per_tile, d2), jnp.float32)
    W2p = jnp.pad(W2, ((0, 0), (0, d2 - ncls)))

    R = 640
    grid = (n // R,)

    degp = _make_deg_kernel(n, e)(col_e, ones_deg, zeros_deg)

    hs, dinv = pl.pallas_call(
        _mm1_body,
        grid=grid,
        in_specs=[
            pl.BlockSpec((R, d), lambda i: (i, 0)),
            pl.BlockSpec((d, hdim), lambda i: (0, 0)),
            pl.BlockSpec((NC, R, DEG_W), lambda i: (0, i, 0)),
        ],
        out_specs=[
            pl.BlockSpec((R, hdim), lambda i: (i, 0)),
            pl.BlockSpec((R, 8), lambda i: (i, 0)),
        ],
        out_shape=[
            jax.ShapeDtypeStruct((n, hdim), jnp.float32),
            jax.ShapeDtypeStruct((n, 8), jnp.float32),
        ],
    )(x, W1, degp)

    p1 = _make_agg_kernel(n, e, half, True)(
        hs.reshape(2 * n, half), row_f2, col_f, zeros_2)

    h2s = pl.pallas_call(
        _mid_body,
        grid=grid,
        in_specs=[
            pl.BlockSpec((R, hdim), lambda i: (i, 0)),
            pl.BlockSpec((R, hdim), lambda i: (i, 0)),
            pl.BlockSpec((R, 8), lambda i: (i, 0)),
            pl.BlockSpec((1, hdim), lambda i: (0, 0)),
            pl.BlockSpec((1, hdim), lambda i: (0, 0)),
            pl.BlockSpec((1, hdim), lambda i: (0, 0)),
            pl.BlockSpec((hdim, d2), lambda i: (0, 0)),
        ],
        out_specs=pl.BlockSpec((R, 2 * d2), lambda i: (i, 0)),
        out_shape=jax.ShapeDtypeStruct((n, 2 * d2), jnp.float32),
    )(p1, hs, dinv, b1.reshape(1, hdim), gamma1.reshape(1, hdim),
      beta1.reshape(1, hdim), W2p)

    p2 = _make_agg_kernel(n, e, d2, False)(
        h2s.reshape(2 * n, d2), row_e2, col_e, zeros_2)

    Rf = 1000
    out = pl.pallas_call(
        _final_body,
        grid=(n0 // Rf,),
        in_specs=[
            pl.BlockSpec((Rf, 2 * d2), lambda i: (i, 0)),
            pl.BlockSpec((Rf, 2 * d2), lambda i: (i, 0)),
            pl.BlockSpec((Rf, 8), lambda i: (i, 0)),
            pl.BlockSpec((1, ncls), lambda i: (0, 0)),
        ],
        out_specs=pl.BlockSpec((Rf, ncls), lambda i: (i, 0)),
        out_shape=jax.ShapeDtypeStruct((n0, ncls), jnp.float32),
    )(p2, h2s, dinv, b2.reshape(1, ncls))

    return out


# speedup vs baseline: 36.1646x; 1.0225x over previous
import functools

import jax
import jax.numpy as jnp
from jax import lax
from jax.experimental import pallas as pl
from jax.experimental.pallas import tpu as pltpu
from jax.experimental.pallas import tpu_sc as plsc

NC = 2
NS = 16
NW = NC * NS

K = 100
DEG_W = 8


def _sc_mesh():
    return plsc.VectorSubcoreMesh(
        core_axis_name="c", subcore_axis_name="s", num_cores=NC,
        num_subcores=NS)


def _make_deg_kernel(n, e):
    nch = e // (NW * K)
    rows_per_tile = n // NS

    @functools.partial(
        pl.kernel,
        out_type=jax.ShapeDtypeStruct((n, 128), jnp.float32),
        mesh=_sc_mesh(),
        compiler_params=pltpu.CompilerParams(use_tc_tiling_on_sc=False),
        scratch_types=[
            pltpu.VMEM((nch, K), jnp.int32),
            pltpu.VMEM((K, DEG_W), jnp.float32),
            pltpu.VMEM_SHARED((n, DEG_W), jnp.float32),
        ],
    )
    def deg_kernel(col_hbm, ones_hbm, zeros_hbm, out_hbm, idx_c, ones_v,
                   acc):
        cid = lax.axis_index("c")
        sid = lax.axis_index("s")
        wid = sid * NC + cid
        pltpu.sync_copy(col_hbm.at[wid], idx_c)
        pltpu.sync_copy(ones_hbm, ones_v)
        sl = pl.ds(sid * rows_per_tile, rows_per_tile)
        pltpu.sync_copy(zeros_hbm, acc.at[sl])
        plsc.subcore_barrier()

        def step(j, carry):
            pltpu.sync_copy(ones_v, acc.at[idx_c.at[j]], add=True)
            return carry

        lax.fori_loop(0, nch, step, 0)
        plsc.subcore_barrier()
        pltpu.sync_copy(acc.at[sl],
                        out_hbm.at[sl, pl.ds(cid * DEG_W, DEG_W)])

    return deg_kernel


def _make_agg_kernel(n, e, d, feature_split):
    tiles = NS if feature_split else NW
    nch = e // (tiles * K)
    rows_per_tile = n // NS

    @functools.partial(
        pl.kernel,
        out_type=jax.ShapeDtypeStruct((n, 2 * d), jnp.float32),
        mesh=_sc_mesh(),
        compiler_params=pltpu.CompilerParams(use_tc_tiling_on_sc=False),
        scratch_types=(
            [pltpu.VMEM((nch // 2, K), jnp.int32)] * 2
            + [pltpu.VMEM((K, d), jnp.float32)] * 10
            + [pltpu.VMEM_SHARED((n, d), jnp.float32)]
            + [pltpu.SemaphoreType.DMA] * 4
        ),
    )
    def agg_kernel(h_hbm, row_hbm, col_hbm, zeros_hbm, out_hbm, idx_r,
                   idx_c, a0, a1, a2, a3, a4, b0, b1, b2, b3, b4, acc,
                   sga, sgb, ssa, ssb):
        cid = lax.axis_index("c")
        sid = lax.axis_index("s")
        if feature_split:
            src_hbm = h_hbm.at[pl.ds(cid, h_hbm.shape[0] - 1)]
            tid = sid
        else:
            src_hbm = h_hbm.at[pl.ds(0, h_hbm.shape[0] - 1)]
            tid = sid * NC + cid
        rows_slab = row_hbm.at[tid]
        cols_slab = col_hbm.at[tid]
        sl = pl.ds(sid * rows_per_tile, rows_per_tile)
        pltpu.sync_copy(zeros_hbm, acc.at[sl])
        plsc.subcore_barrier()

        bufa = (a0, a1, a2, a3, a4)
        bufb = (b0, b1, b2, b3, b4)
        G = 5
        nch2 = nch // 2

        def fire_g(base, bufs, sem):
            for t in range(G):
                pltpu.async_copy(src_hbm.at[idx_r.at[base + t]], bufs[t],
                                 sem)

        def drain_g(base, bufs, sem):
            for t in range(G):
                pltpu.make_async_copy(src_hbm.at[idx_r.at[base + t]],
                                      bufs[t], sem).wait()

        def fire_s(base, bufs, sem):
            for t in range(G):
                pltpu.async_copy(bufs[t], acc.at[idx_c.at[base + t]], sem,
                                 add=True)

        def drain_s(base, bufs, sem):
            for t in range(G):
                pltpu.make_async_copy(bufs[t], acc.at[idx_c.at[base + t]],
                                      sem).wait()

        niter = nch2 // (2 * G)
        for ph in range(2):
            pltpu.sync_copy(rows_slab.at[pl.ds(ph * nch2, nch2)], idx_r)
            pltpu.sync_copy(cols_slab.at[pl.ds(ph * nch2, nch2)], idx_c)
            fire_g(0, bufa, sga)

            def step(i, carry):
                base = i * 2 * G
                drain_g(base, bufa, sga)

                @pl.when(i > 0)
                def _():
                    drain_s(base - G, bufb, ssb)

                fire_g(base + G, bufb, sgb)
                fire_s(base, bufa, ssa)
                drain_g(base + G, bufb, sgb)
                drain_s(base, bufa, ssa)

                @pl.when(i < niter - 1)
                def _():
                    fire_g(base + 2 * G, bufa, sga)

                fire_s(base + G, bufb, ssb)
                return carry

            lax.fori_loop(0, niter, step, 0)
            drain_s(niter * 2 * G - G, bufb, ssb)

        plsc.subcore_barrier()
        pltpu.sync_copy(acc.at[sl], out_hbm.at[sl, pl.ds(cid * d, d)])

    return agg_kernel


def _mm1_body(x_ref, w_ref, degp_ref, hs_ref, dinv_ref):
    deg = degp_ref[:, 0:1] + degp_ref[:, DEG_W:DEG_W + 1] + 1.0
    dinv = lax.rsqrt(deg)
    h = jnp.dot(x_ref[...], w_ref[...], preferred_element_type=jnp.float32)
    hs_ref[...] = dinv * h
    dinv_ref[...] = jnp.broadcast_to(dinv, dinv_ref.shape)


def _mid_body(p_ref, hs_ref, dinv_ref, b1_ref, g_ref, be_ref, w2_ref,
              h2s_ref):
    dinv = dinv_ref[:, 0:1]
    t = dinv * (p_ref[...] + hs_ref[...]) + b1_ref[...]
    m = jnp.mean(t, axis=-1, keepdims=True)
    v = jnp.mean((t - m) ** 2, axis=-1, keepdims=True)
    t = (t - m) * lax.rsqrt(v + 1e-6) * g_ref[...] + be_ref[...]
    a = jnp.maximum(t, 0.0)
    h2 = jnp.dot(a, w2_ref[...], preferred_element_type=jnp.float32)
    s2 = dinv * h2
    h2s_ref[...] = jnp.concatenate([s2, s2], axis=-1)


def _final_body(p_ref, h2s_ref, dinv_ref, b2_ref, out_ref):
    dinv = dinv_ref[:, 0:1]
    d = p_ref.shape[-1] // 2
    ncls = out_ref.shape[-1]
    psum = p_ref[:, :d] + p_ref[:, d:]
    t = dinv * (psum + h2s_ref[:, :d])
    out_ref[...] = t[:, :ncls] + b2_ref[...]


def kernel(x, edge_index, edge_weight, W1, b1, gamma1, beta1, W2, b2):
    n0, d = x.shape
    hdim = W1.shape[1]
    ncls = W2.shape[1]
    e = edge_index.shape[1]
    n = ((n0 + NS * 8 - 1) // (NS * 8)) * (NS * 8)
    n = max(n, 10240)
    rows_per_tile = n // NS
    d2 = 64
    half = hdim // 2

    row = edge_index[0]
    col = edge_index[1]
    row2 = row * 2
    nch_e = e // (NW * K)
    row_e2 = row2.reshape(NW, nch_e, K)
    col_e = col.reshape(NW, nch_e, K)
    nch_f = e // (NS * K)
    row_f2 = row2.reshape(NS, nch_f, K)
    col_f = col.reshape(NS, nch_f, K)

    ones_deg = jnp.ones((K, DEG_W), jnp.float32)
    zeros_deg = jnp.zeros((rows_per_tile, DEG_W), jnp.float32)
    zeros_2 = jnp.zeros((rows_per_tile, d2), jnp.float32)
    W2p = jnp.pad(W2, ((0, 0), (0, d2 - ncls)))

    R = 640
    grid = (n // R,)

    degp = _make_deg_kernel(n, e)(col_e, ones_deg, zeros_deg)

    hs, dinv = pl.pallas_call(
        _mm1_body,
        grid=grid,
        in_specs=[
            pl.BlockSpec((R, d), lambda i: (i, 0)),
            pl.BlockSpec((d, hdim), lambda i: (0, 0)),
            pl.BlockSpec((R, 128), lambda i: (i, 0)),
        ],
        out_specs=[
            pl.BlockSpec((R, hdim), lambda i: (i, 0)),
            pl.BlockSpec((R, 8), lambda i: (i, 0)),
        ],
        out_shape=[
            jax.ShapeDtypeStruct((n, hdim), jnp.float32),
            jax.ShapeDtypeStruct((n, 8), jnp.float32),
        ],
    )(x, W1, degp)

    p1 = _make_agg_kernel(n, e, half, True)(
        hs.reshape(2 * n, half), row_f2, col_f, zeros_2)

    h2s = pl.pallas_call(
        _mid_body,
        grid=grid,
        in_specs=[
            pl.BlockSpec((R, hdim), lambda i: (i, 0)),
            pl.BlockSpec((R, hdim), lambda i: (i, 0)),
            pl.BlockSpec((R, 8), lambda i: (i, 0)),
            pl.BlockSpec((1, hdim), lambda i: (0, 0)),
            pl.BlockSpec((1, hdim), lambda i: (0, 0)),
            pl.BlockSpec((1, hdim), lambda i: (0, 0)),
            pl.BlockSpec((hdim, d2), lambda i: (0, 0)),
        ],
        out_specs=pl.BlockSpec((R, 2 * d2), lambda i: (i, 0)),
        out_shape=jax.ShapeDtypeStruct((n, 2 * d2), jnp.float32),
    )(p1, hs, dinv, b1.reshape(1, hdim), gamma1.reshape(1, hdim),
      beta1.reshape(1, hdim), W2p)

    p2 = _make_agg_kernel(n, e, d2, False)(
        h2s.reshape(2 * n, d2), row_e2, col_e, zeros_2)

    Rf = 1000
    out = pl.pallas_call(
        _final_body,
        grid=(n0 // Rf,),
        in_specs=[
            pl.BlockSpec((Rf, 2 * d2), lambda i: (i, 0)),
            pl.BlockSpec((Rf, 2 * d2), lambda i: (i, 0)),
            pl.BlockSpec((Rf, 8), lambda i: (i, 0)),
            pl.BlockSpec((1, ncls), lambda i: (0, 0)),
        ],
        out_specs=pl.BlockSpec((Rf, ncls), lambda i: (i, 0)),
        out_shape=jax.ShapeDtypeStruct((n0, ncls), jnp.float32),
    )(p2, h2s, dinv, b2.reshape(1, ncls))

    return out
